# baseline (speedup 1.0000x reference)
import functools

import jax
import jax.numpy as jnp
from jax import lax
from jax.experimental import pallas as pl
from jax.experimental.pallas import tpu as pltpu
from jax.experimental.pallas import tpu_sc as plsc

_N = 10000
_E = 320000
_D = 128
_G = 64
_NB = 16
_NO = 128
_NP = 10240
_NC = 2
_NS = 16
_C = 128
_NCH = 160
_NBUF = 2
_DH = 64
_EPAD = _NS * _NCH * _C
_BLK = 1024


def _make_sc_apply():
    mesh = plsc.VectorSubcoreMesh(
        core_axis_name="c", subcore_axis_name="s",
        num_cores=_NC, num_subcores=_NS)
    rows_per = _NP // _NS
    n_cp = rows_per // _C

    def body(u_hbm, gidx_hbm, sidx_hbm, out_hbm, gidx_v, sidx_v,
             buf0, buf1, y_sh, *sems):
        bufs = (buf0, buf1)
        gsem = sems[:_NBUF]
        ssem = sems[_NBUF:]
        cid = lax.axis_index("c")
        sid = lax.axis_index("s")
        pltpu.sync_copy(gidx_hbm.at[sid], gidx_v)
        pltpu.sync_copy(sidx_hbm.at[sid], sidx_v)

        zero = jnp.zeros((16,), jnp.float32)

        def zrow(i, carry):
            for j in range(_D // 16):
                buf0[i, pl.ds(j * 16, 16)] = zero
            return carry

        lax.fori_loop(0, _C, zrow, 0)
        base = sid * rows_per

        plsc.subcore_barrier()

        for b in range(_NBUF):
            pltpu.async_copy(
                u_hbm.at[gidx_v.at[b]], bufs[b], gsem[b])

        def grp(i, carry):
            for b in range(_NBUF):
                j = _NBUF * i + b
                jn = jnp.minimum(j + _NBUF, _NCH - 1)
                pltpu.make_async_copy(
                    u_hbm.at[gidx_v.at[j]], bufs[b], gsem[b]).wait()
                pltpu.async_copy(
                    u_hbm.at[gidx_v.at[jn]], bufs[b], gsem[b])
            return carry

        lax.fori_loop(0, _NCH // _NBUF, grp, 0)
        for b in range(_NBUF):
            pltpu.make_async_copy(
                u_hbm.at[gidx_v.at[_NCH - 1]], bufs[b], gsem[b]).wait()
        plsc.subcore_barrier()

        def ocp(i, carry):
            pltpu.sync_copy(y_sh.at[pl.ds(base + i * _C, _C)],
                            out_hbm.at[cid, pl.ds(base + i * _C, _C)])
            return carry

        lax.fori_loop(0, n_cp, ocp, 0)

    return pl.kernel(
        body,
        out_type=jax.ShapeDtypeStruct((_NC, _NP, _DH), jnp.float32),
        mesh=mesh,
        scratch_types=[
            pltpu.VMEM((_NCH, _C), jnp.int32),
            pltpu.VMEM((_NCH, _C), jnp.int32),
            pltpu.VMEM((_C, _D), jnp.float32),
            pltpu.VMEM((_C, _D), jnp.float32),
            pltpu.VMEM_SHARED((_NP, _DH), jnp.float32),
        ] + [pltpu.SemaphoreType.DMA] * (2 * _NBUF),
        compiler_params=pltpu.CompilerParams(use_tc_tiling_on_sc=False),
        name="sc_adj_apply",
    )


_sc_apply = _make_sc_apply()


def _dinv(degc):
    def body(a_ref, o_ref):
        deg = a_ref[...]
        node = (lax.broadcasted_iota(jnp.int32, (80, 128), 0) * 128
                + lax.broadcasted_iota(jnp.int32, (80, 128), 1))
        ok = (deg > 0.5) & (node < _N)
        o_ref[...] = jnp.where(ok, lax.rsqrt(jnp.maximum(deg, 1.0)), 0.0)

    return pl.pallas_call(
        body,
        out_shape=jax.ShapeDtypeStruct((80, 128), jnp.float32),
    )(degc)


def _step0(xin, dinvb, W, F):
    def body(x_ref, d_ref, w_ref, u_ref, a_ref):
        xv = x_ref[...]
        uu = d_ref[...] * xv
        u_ref[0] = uu[:, :_DH]
        u_ref[1] = uu[:, _DH:]
        a_ref[...] = jnp.dot(xv, w_ref[...], preferred_element_type=jnp.float32)

    return pl.pallas_call(
        body,
        grid=(_NP // _BLK,),
        in_specs=[
            pl.BlockSpec((_BLK, _D), lambda i: (i, 0)),
            pl.BlockSpec((_BLK, _D), lambda i: (i, 0)),
            pl.BlockSpec((_D, F), lambda i: (0, 0)),
        ],
        out_specs=[
            pl.BlockSpec((_NC, _BLK, _DH), lambda i: (0, i, 0)),
            pl.BlockSpec((_BLK, F), lambda i: (i, 0)),
        ],
        out_shape=[
            jax.ShapeDtypeStruct((_NC, _NP, _DH), jnp.float32),
            jax.ShapeDtypeStruct((_NP, F), jnp.float32),
        ],
    )(xin, dinvb, W)


def _stepk(z, dinvb, txprev, W, acc, c1, c2, F):
    def body(z_ref, d_ref, p_ref, w_ref, ain_ref,
             tx_ref, u_ref, aout_ref):
        zz = jnp.concatenate([z_ref[0], z_ref[1]], axis=1)
        tx = c1 * d_ref[...] * zz + c2 * p_ref[...]
        tx_ref[...] = tx
        uu = d_ref[...] * tx
        u_ref[0] = uu[:, :_DH]
        u_ref[1] = uu[:, _DH:]
        aout_ref[...] = ain_ref[...] + jnp.dot(
            tx, w_ref[...], preferred_element_type=jnp.float32)

    return pl.pallas_call(
        body,
        grid=(_NP // _BLK,),
        in_specs=[
            pl.BlockSpec((_NC, _BLK, _DH), lambda i: (0, i, 0)),
            pl.BlockSpec((_BLK, _D), lambda i: (i, 0)),
            pl.BlockSpec((_BLK, _D), lambda i: (i, 0)),
            pl.BlockSpec((_D, F), lambda i: (0, 0)),
            pl.BlockSpec((_BLK, F), lambda i: (i, 0)),
        ],
        out_specs=[
            pl.BlockSpec((_BLK, _D), lambda i: (i, 0)),
            pl.BlockSpec((_NC, _BLK, _DH), lambda i: (0, i, 0)),
            pl.BlockSpec((_BLK, F), lambda i: (i, 0)),
        ],
        out_shape=[
            jax.ShapeDtypeStruct((_NP, _D), jnp.float32),
            jax.ShapeDtypeStruct((_NC, _NP, _DH), jnp.float32),
            jax.ShapeDtypeStruct((_NP, F), jnp.float32),
        ],
    )(z, dinvb, txprev, W, acc)


def _steplast(z, dinvb, txprev, W, b, acc, F):
    def body(z_ref, d_ref, p_ref, w_ref, b_ref, ain_ref, o_ref):
        zz = jnp.concatenate([z_ref[0], z_ref[1]], axis=1)
        tx = -2.0 * d_ref[...] * zz - p_ref[...]
        o_ref[...] = jnp.maximum(
            ain_ref[...]
            + jnp.dot(tx, w_ref[...], preferred_element_type=jnp.float32)
            + b_ref[...], 0.0)

    return pl.pallas_call(
        body,
        grid=(_NP // _BLK,),
        in_specs=[
            pl.BlockSpec((_NC, _BLK, _DH), lambda i: (0, i, 0)),
            pl.BlockSpec((_BLK, _D), lambda i: (i, 0)),
            pl.BlockSpec((_BLK, _D), lambda i: (i, 0)),
            pl.BlockSpec((_D, F), lambda i: (0, 0)),
            pl.BlockSpec((1, F), lambda i: (0, 0)),
            pl.BlockSpec((_BLK, F), lambda i: (i, 0)),
        ],
        out_specs=pl.BlockSpec((_BLK, F), lambda i: (i, 0)),
        out_shape=jax.ShapeDtypeStruct((_NP, F), jnp.float32),
    )(z, dinvb, txprev, W, b, acc)


def _pool(H, batchf):
    def body(b_ref, h_ref, s_ref, c_ref):
        i = pl.program_id(0)
        oh = (b_ref[...] == lax.broadcasted_iota(
            jnp.int32, (_BLK, _G), 1).astype(jnp.float32)).astype(jnp.float32)
        psum = lax.dot_general(oh, h_ref[...], (((0,), (0,)), ((), ())),
                               preferred_element_type=jnp.float32)
        pcnt = jnp.broadcast_to(jnp.sum(oh, axis=0)[:, None], (_G, 128))

        @pl.when(i == 0)
        def _():
            s_ref[...] = jnp.zeros_like(s_ref)
            c_ref[...] = jnp.zeros_like(c_ref)

        s_ref[...] += psum
        c_ref[...] += pcnt

    return pl.pallas_call(
        body,
        grid=(_NP // _BLK,),
        in_specs=[
            pl.BlockSpec((_BLK, 1), lambda i: (i, 0)),
            pl.BlockSpec((_BLK, 512), lambda i: (i, 0)),
        ],
        out_specs=[
            pl.BlockSpec((_G, 512), lambda i: (0, 0)),
            pl.BlockSpec((_G, 128), lambda i: (0, 0)),
        ],
        out_shape=[
            jax.ShapeDtypeStruct((_G, 512), jnp.float32),
            jax.ShapeDtypeStruct((_G, 128), jnp.float32),
        ],
    )(batchf, H)


def _head(sums, cnt, Wfc, bfc, cmf):
    def body(s_ref, c_ref, w_ref, b_ref, cm_ref, o_ref):
        counts = jnp.maximum(c_ref[...][:, 0:1], 1.0)
        pooled = s_ref[...] / counts
        logits = jnp.dot(pooled, w_ref[...],
                         preferred_element_type=jnp.float32) + b_ref[...]
        cmcol = jnp.reshape(cm_ref[...], (_NO, 1))
        P = (cmcol == lax.broadcasted_iota(
            jnp.int32, (_NO, _NB), 1).astype(jnp.float32)).astype(jnp.float32)
        seg = jnp.log(jnp.dot(jnp.exp(logits), P,
                              preferred_element_type=jnp.float32))
        norm = lax.dot_general(seg, P, (((1,), (1,)), ((), ())),
                               preferred_element_type=jnp.float32)
        o_ref[...] = logits - norm

    return pl.pallas_call(
        body,
        out_shape=jax.ShapeDtypeStruct((_G, _NO), jnp.float32),
    )(sums, cnt, Wfc, bfc, cmf)


def _conv(xin, dinvb, gidx, sidx, W, b2, F):
    u0, acc = _step0(xin, dinvb, W[0], F)
    z = _sc_apply(u0.reshape(_NP, _D), gidx, sidx)
    tx1, u1, acc = _stepk(z, dinvb, xin, W[1], acc, -1.0, 0.0, F)
    z = _sc_apply(u1.reshape(_NP, _D), gidx, sidx)
    tx2, u2, acc = _stepk(z, dinvb, xin, W[2], acc, -2.0, -1.0, F)
    z = _sc_apply(u2.reshape(_NP, _D), gidx, sidx)
    tx3, u3, acc = _stepk(z, dinvb, tx1, W[3], acc, -2.0, -1.0, F)
    z = _sc_apply(u3.reshape(_NP, _D), gidx, sidx)
    return _steplast(z, dinvb, tx2, W[4], b2, acc, F)


def kernel(x, edge_index, batch, class_mask,
           W11, b11, W12, b12, W21, b21, W22, b22, Wfc, bfc):
    f32 = jnp.float32
    xp = jnp.pad(x, ((0, _NP - _N), (0, 0)))

    pad = _EPAD - _E
    sink = jnp.full((pad,), _NP - 1, jnp.int32)
    g_f = jnp.concatenate([edge_index[0], sink]).reshape(_NS, _NCH, _C)
    s_f = jnp.concatenate([edge_index[1], sink]).reshape(_NS, _NCH, _C)
    g_f2 = jnp.stack([g_f, g_f + _NP])
    s_f2 = jnp.stack([s_f, s_f + _NP])

    ones = jnp.ones((_NP, _D), f32)
    z_cnt_r = _sc_apply(ones, g_f, s_f)
    z_cnt_f = _sc_apply(ones, s_f, g_f)
    d_f = _dinv(z_cnt_f[0, :, 0].reshape(80, 128))
    d_r = _dinv(z_cnt_r[0, :, 0].reshape(80, 128))
    dinvb_f = jnp.broadcast_to(d_f.reshape(_NP, 1), (_NP, _D))
    dinvb_r = jnp.broadcast_to(d_r.reshape(_NP, 1), (_NP, _D))

    x1 = _conv(xp, dinvb_f, g_f, s_f, W11, b11.reshape(1, 64), 64)
    x2 = _conv(xp, dinvb_r, s_f, g_f, W12, b12.reshape(1, 64), 64)
    h = jnp.concatenate([x1, x2], axis=1)
    y1 = _conv(h, dinvb_f, g_f, s_f, W21, b21.reshape(1, 256), 256)
    y2 = _conv(h, dinvb_r, s_f, g_f, W22, b22.reshape(1, 256), 256)
    H = jnp.concatenate([y1, y2], axis=1)

    batchf = jnp.pad(batch, (0, _NP - _N), constant_values=_G)
    batchf = batchf.astype(f32).reshape(_NP, 1)
    sums, cnt = _pool(H, batchf)
    return _head(sums, cnt, Wfc, bfc.reshape(1, _NO),
                 class_mask.astype(f32).reshape(1, _NO))


# speedup vs baseline: 2.6897x; 2.6897x over previous
import jax
import jax.numpy as jnp
from jax import lax
from jax.experimental import pallas as pl
from jax.experimental.pallas import tpu as pltpu
from jax.experimental.pallas import tpu_sc as plsc

_N = 10000
_E = 320000
_D = 128
_G = 64
_NB = 16
_NO = 128
_NP = 10240
_NC = 2
_NS = 16
_C = 128
_NCH = 160
_NBUF = 4
_DH = 64
_EPAD = _NS * _NCH * _C
_BLK = 1024

_SC_PARAMS = pltpu.CompilerParams(use_tc_tiling_on_sc=False)


def _make_sc_apply():
    mesh = plsc.VectorSubcoreMesh(
        core_axis_name="c", subcore_axis_name="s",
        num_cores=_NC, num_subcores=_NS)
    rows_per = _NP // _NS
    n_cp = rows_per // _C

    def body(u_hbm, gidx_hbm, sidx_hbm, out_hbm, gidx_v, sidx_v,
             buf0, buf1, buf2, buf3, y_sh, *sems):
        bufs = (buf0, buf1, buf2, buf3)
        gsem = sems[:_NBUF]
        ssem = sems[_NBUF:]
        cid = lax.axis_index("c")
        sid = lax.axis_index("s")
        pltpu.sync_copy(gidx_hbm.at[cid, sid], gidx_v)
        pltpu.sync_copy(sidx_hbm.at[sid], sidx_v)

        zero = jnp.zeros((16,), jnp.float32)

        def zrow(i, carry):
            for j in range(_DH // 16):
                buf0[i, pl.ds(j * 16, 16)] = zero
            return carry

        lax.fori_loop(0, _C, zrow, 0)
        base = sid * rows_per

        def zcp(i, carry):
            pltpu.sync_copy(buf0, y_sh.at[pl.ds(base + i * _C, _C)])
            return carry

        lax.fori_loop(0, n_cp, zcp, 0)
        plsc.subcore_barrier()

        for b in range(_NBUF):
            pltpu.async_copy(u_hbm.at[gidx_v.at[b]], bufs[b], gsem[b])

        def grp(i, carry):
            for b in range(_NBUF):
                j = _NBUF * i + b
                pltpu.make_async_copy(
                    u_hbm.at[gidx_v.at[j]], bufs[b], gsem[b]).wait()
                pltpu.async_copy(
                    bufs[b], y_sh.at[sidx_v.at[j]], ssem[b], add=True)
            for b in range(_NBUF):
                j = _NBUF * i + b
                jn = jnp.minimum(j + _NBUF, _NCH - 1)
                pltpu.make_async_copy(
                    bufs[b], y_sh.at[sidx_v.at[j]], ssem[b]).wait()
                pltpu.async_copy(u_hbm.at[gidx_v.at[jn]], bufs[b], gsem[b])
            return carry

        lax.fori_loop(0, _NCH // _NBUF, grp, 0)
        for b in range(_NBUF):
            pltpu.make_async_copy(
                u_hbm.at[gidx_v.at[_NCH - 1]], bufs[b], gsem[b]).wait()
        plsc.subcore_barrier()

        def ocp(i, carry):
            pltpu.sync_copy(y_sh.at[pl.ds(base + i * _C, _C)],
                            out_hbm.at[cid, pl.ds(base + i * _C, _C)])
            return carry

        lax.fori_loop(0, n_cp, ocp, 0)

    return pl.kernel(
        body,
        out_type=jax.ShapeDtypeStruct((_NC, _NP, _DH), jnp.float32),
        mesh=mesh,
        scratch_types=[
            pltpu.VMEM((_NCH, _C), jnp.int32),
            pltpu.VMEM((_NCH, _C), jnp.int32),
            pltpu.VMEM((_C, _DH), jnp.float32),
            pltpu.VMEM((_C, _DH), jnp.float32),
            pltpu.VMEM((_C, _DH), jnp.float32),
            pltpu.VMEM((_C, _DH), jnp.float32),
            pltpu.VMEM_SHARED((_NP, _DH), jnp.float32),
        ] + [pltpu.SemaphoreType.DMA] * (2 * _NBUF),
        compiler_params=_SC_PARAMS,
        name="sc_adj_apply",
    )


def _make_sc_deg():
    mesh = plsc.VectorSubcoreMesh(
        core_axis_name="c", subcore_axis_name="s",
        num_cores=_NC, num_subcores=_NS)
    _W = 16
    rows_per = _NP // _NS
    n_cp = rows_per // _C

    def body(idx_hbm, out_hbm, idx_v, buf, y_sh, *sems):
        cid = lax.axis_index("c")
        sid = lax.axis_index("s")
        pltpu.sync_copy(idx_hbm.at[cid, sid], idx_v)

        one = jnp.ones((16,), jnp.float32)
        zero = jnp.zeros((16,), jnp.float32)

        def orow(i, carry):
            buf[i, pl.ds(0, 16)] = one
            return carry

        lax.fori_loop(0, _C, orow, 0)
        base = sid * rows_per

        def zrow(i, carry):
            buf[_C + i, pl.ds(0, 16)] = zero
            return carry

        lax.fori_loop(0, _C, zrow, 0)

        def zcp(i, carry):
            pltpu.sync_copy(buf.at[pl.ds(_C, _C)],
                            y_sh.at[pl.ds(base + i * _C, _C)])
            return carry

        lax.fori_loop(0, n_cp, zcp, 0)
        plsc.subcore_barrier()

        ones_src = buf.at[pl.ds(0, _C)]

        def grp(i, carry):
            for b in range(_NBUF):
                j = _NBUF * i + b
                pltpu.async_copy(
                    ones_src, y_sh.at[idx_v.at[j]], sems[b], add=True)
            for b in range(_NBUF):
                j = _NBUF * i + b
                pltpu.make_async_copy(
                    ones_src, y_sh.at[idx_v.at[j]], sems[b]).wait()
            return carry

        lax.fori_loop(0, _NCH // _NBUF, grp, 0)
        plsc.subcore_barrier()

        def ocp(i, carry):
            pltpu.sync_copy(y_sh.at[pl.ds(base + i * _C, _C)],
                            out_hbm.at[cid, pl.ds(base + i * _C, _C)])
            return carry

        lax.fori_loop(0, n_cp, ocp, 0)

    return pl.kernel(
        body,
        out_type=jax.ShapeDtypeStruct((_NC, _NP, _W), jnp.float32),
        mesh=mesh,
        scratch_types=[
            pltpu.VMEM((_NCH, _C), jnp.int32),
            pltpu.VMEM((2 * _C, _W), jnp.float32),
            pltpu.VMEM_SHARED((_NP, _W), jnp.float32),
        ] + [pltpu.SemaphoreType.DMA] * _NBUF,
        compiler_params=_SC_PARAMS,
        name="sc_deg",
    )


_sc_apply = _make_sc_apply()
_sc_deg = _make_sc_deg()


def _dinv(degc):
    def body(a_ref, o_ref):
        deg = a_ref[...]
        node = (lax.broadcasted_iota(jnp.int32, (80, 128), 0) * 128
                + lax.broadcasted_iota(jnp.int32, (80, 128), 1))
        ok = (deg > 0.5) & (node < _N)
        o_ref[...] = jnp.where(ok, lax.rsqrt(jnp.maximum(deg, 1.0)), 0.0)

    return pl.pallas_call(
        body,
        out_shape=jax.ShapeDtypeStruct((80, 128), jnp.float32),
    )(degc)


def _step0(xin, dinvb, W, F):
    def body(x_ref, d_ref, w_ref, u_ref, a_ref):
        xv = x_ref[...]
        uu = d_ref[...] * xv
        u_ref[0] = uu[:, :_DH]
        u_ref[1] = uu[:, _DH:]
        a_ref[...] = jnp.dot(xv, w_ref[...], preferred_element_type=jnp.float32)

    return pl.pallas_call(
        body,
        grid=(_NP // _BLK,),
        in_specs=[
            pl.BlockSpec((_BLK, _D), lambda i: (i, 0)),
            pl.BlockSpec((_BLK, _D), lambda i: (i, 0)),
            pl.BlockSpec((_D, F), lambda i: (0, 0)),
        ],
        out_specs=[
            pl.BlockSpec((_NC, _BLK, _DH), lambda i: (0, i, 0)),
            pl.BlockSpec((_BLK, F), lambda i: (i, 0)),
        ],
        out_shape=[
            jax.ShapeDtypeStruct((_NC, _NP, _DH), jnp.float32),
            jax.ShapeDtypeStruct((_NP, F), jnp.float32),
        ],
    )(xin, dinvb, W)


def _stepk(z, dinvb, txprev, W, acc, c1, c2, F):
    def body(z_ref, d_ref, p_ref, w_ref, ain_ref,
             tx_ref, u_ref, aout_ref):
        zz = jnp.concatenate([z_ref[0], z_ref[1]], axis=1)
        tx = c1 * d_ref[...] * zz + c2 * p_ref[...]
        tx_ref[...] = tx
        uu = d_ref[...] * tx
        u_ref[0] = uu[:, :_DH]
        u_ref[1] = uu[:, _DH:]
        aout_ref[...] = ain_ref[...] + jnp.dot(
            tx, w_ref[...], preferred_element_type=jnp.float32)

    return pl.pallas_call(
        body,
        grid=(_NP // _BLK,),
        in_specs=[
            pl.BlockSpec((_NC, _BLK, _DH), lambda i: (0, i, 0)),
            pl.BlockSpec((_BLK, _D), lambda i: (i, 0)),
            pl.BlockSpec((_BLK, _D), lambda i: (i, 0)),
            pl.BlockSpec((_D, F), lambda i: (0, 0)),
            pl.BlockSpec((_BLK, F), lambda i: (i, 0)),
        ],
        out_specs=[
            pl.BlockSpec((_BLK, _D), lambda i: (i, 0)),
            pl.BlockSpec((_NC, _BLK, _DH), lambda i: (0, i, 0)),
            pl.BlockSpec((_BLK, F), lambda i: (i, 0)),
        ],
        out_shape=[
            jax.ShapeDtypeStruct((_NP, _D), jnp.float32),
            jax.ShapeDtypeStruct((_NC, _NP, _DH), jnp.float32),
            jax.ShapeDtypeStruct((_NP, F), jnp.float32),
        ],
    )(z, dinvb, txprev, W, acc)


def _steplast(z, dinvb, txprev, W, b, acc, F):
    def body(z_ref, d_ref, p_ref, w_ref, b_ref, ain_ref, o_ref):
        zz = jnp.concatenate([z_ref[0], z_ref[1]], axis=1)
        tx = -2.0 * d_ref[...] * zz - p_ref[...]
        o_ref[...] = jnp.maximum(
            ain_ref[...]
            + jnp.dot(tx, w_ref[...], preferred_element_type=jnp.float32)
            + b_ref[...], 0.0)

    return pl.pallas_call(
        body,
        grid=(_NP // _BLK,),
        in_specs=[
            pl.BlockSpec((_NC, _BLK, _DH), lambda i: (0, i, 0)),
            pl.BlockSpec((_BLK, _D), lambda i: (i, 0)),
            pl.BlockSpec((_BLK, _D), lambda i: (i, 0)),
            pl.BlockSpec((_D, F), lambda i: (0, 0)),
            pl.BlockSpec((1, F), lambda i: (0, 0)),
            pl.BlockSpec((_BLK, F), lambda i: (i, 0)),
        ],
        out_specs=pl.BlockSpec((_BLK, F), lambda i: (i, 0)),
        out_shape=jax.ShapeDtypeStruct((_NP, F), jnp.float32),
    )(z, dinvb, txprev, W, b, acc)


def _pool(H, batchf):
    def body(b_ref, h_ref, s_ref, c_ref):
        i = pl.program_id(0)
        oh = (b_ref[...] == lax.broadcasted_iota(
            jnp.int32, (_BLK, _G), 1).astype(jnp.float32)).astype(jnp.float32)
        psum = lax.dot_general(oh, h_ref[...], (((0,), (0,)), ((), ())),
                               preferred_element_type=jnp.float32)
        pcnt = jnp.broadcast_to(jnp.sum(oh, axis=0)[:, None], (_G, 128))

        @pl.when(i == 0)
        def _():
            s_ref[...] = jnp.zeros_like(s_ref)
            c_ref[...] = jnp.zeros_like(c_ref)

        s_ref[...] += psum
        c_ref[...] += pcnt

    return pl.pallas_call(
        body,
        grid=(_NP // _BLK,),
        in_specs=[
            pl.BlockSpec((_BLK, 1), lambda i: (i, 0)),
            pl.BlockSpec((_BLK, 512), lambda i: (i, 0)),
        ],
        out_specs=[
            pl.BlockSpec((_G, 512), lambda i: (0, 0)),
            pl.BlockSpec((_G, 128), lambda i: (0, 0)),
        ],
        out_shape=[
            jax.ShapeDtypeStruct((_G, 512), jnp.float32),
            jax.ShapeDtypeStruct((_G, 128), jnp.float32),
        ],
    )(batchf, H)


def _head(sums, cnt, Wfc, bfc, cmf):
    def body(s_ref, c_ref, w_ref, b_ref, cm_ref, o_ref):
        counts = jnp.maximum(c_ref[...][:, 0:1], 1.0)
        pooled = s_ref[...] / counts
        logits = jnp.dot(pooled, w_ref[...],
                         preferred_element_type=jnp.float32) + b_ref[...]
        cmcol = jnp.reshape(cm_ref[...], (_NO, 1))
        P = (cmcol == lax.broadcasted_iota(
            jnp.int32, (_NO, _NB), 1).astype(jnp.float32)).astype(jnp.float32)
        seg = jnp.log(jnp.dot(jnp.exp(logits), P,
                              preferred_element_type=jnp.float32))
        norm = lax.dot_general(seg, P, (((1,), (1,)), ((), ())),
                               preferred_element_type=jnp.float32)
        o_ref[...] = logits - norm

    return pl.pallas_call(
        body,
        out_shape=jax.ShapeDtypeStruct((_G, _NO), jnp.float32),
    )(sums, cnt, Wfc, bfc, cmf)


def _conv(xin, dinvb, gidx, sidx, W, b2, F):
    u0, acc = _step0(xin, dinvb, W[0], F)
    z = _sc_apply(u0.reshape(_NC * _NP, _DH), gidx, sidx)
    tx1, u1, acc = _stepk(z, dinvb, xin, W[1], acc, -1.0, 0.0, F)
    z = _sc_apply(u1.reshape(_NC * _NP, _DH), gidx, sidx)
    tx2, u2, acc = _stepk(z, dinvb, xin, W[2], acc, -2.0, -1.0, F)
    z = _sc_apply(u2.reshape(_NC * _NP, _DH), gidx, sidx)
    tx3, u3, acc = _stepk(z, dinvb, tx1, W[3], acc, -2.0, -1.0, F)
    z = _sc_apply(u3.reshape(_NC * _NP, _DH), gidx, sidx)
    return _steplast(z, dinvb, tx2, W[4], b2, acc, F)


def kernel(x, edge_index, batch, class_mask,
           W11, b11, W12, b12, W21, b21, W22, b22, Wfc, bfc):
    f32 = jnp.float32
    xp = jnp.pad(x, ((0, _NP - _N), (0, 0)))

    pad = _EPAD - _E
    sink = jnp.full((pad,), _NP - 1, jnp.int32)
    g_f = jnp.concatenate([edge_index[0], sink]).reshape(_NS, _NCH, _C)
    s_f = jnp.concatenate([edge_index[1], sink]).reshape(_NS, _NCH, _C)
    g_f2 = jnp.stack([g_f, g_f + _NP])
    s_f2 = jnp.stack([s_f, s_f + _NP])

    cnt = _sc_deg(jnp.stack([s_f, g_f]))
    d_f = _dinv(cnt[1, :, 0].reshape(80, 128))
    d_r = _dinv(cnt[0, :, 0].reshape(80, 128))
    dinvb_f = jnp.broadcast_to(d_f.reshape(_NP, 1), (_NP, _D))
    dinvb_r = jnp.broadcast_to(d_r.reshape(_NP, 1), (_NP, _D))

    x1 = _conv(xp, dinvb_f, g_f2, s_f, W11, b11.reshape(1, 64), 64)
    x2 = _conv(xp, dinvb_r, s_f2, g_f, W12, b12.reshape(1, 64), 64)
    h = jnp.concatenate([x1, x2], axis=1)
    y1 = _conv(h, dinvb_f, g_f2, s_f, W21, b21.reshape(1, 256), 256)
    y2 = _conv(h, dinvb_r, s_f2, g_f, W22, b22.reshape(1, 256), 256)
    H = jnp.concatenate([y1, y2], axis=1)

    batchf = jnp.pad(batch, (0, _NP - _N), constant_values=_G)
    batchf = batchf.astype(f32).reshape(_NP, 1)
    sums, cnt2 = _pool(H, batchf)
    return _head(sums, cnt2, Wfc, bfc.reshape(1, _NO),
                 class_mask.astype(f32).reshape(1, _NO))


# speedup vs baseline: 3.3216x; 1.2349x over previous
import jax
import jax.numpy as jnp
from jax import lax
from jax.experimental import pallas as pl
from jax.experimental.pallas import tpu as pltpu
from jax.experimental.pallas import tpu_sc as plsc

_N = 10000
_E = 320000
_D = 128
_G = 64
_NB = 16
_NO = 128
_NP = 10240
_NC = 2
_NS = 16
_C = 128
_NCH = 160
_NBUF = 4
_DH = 64
_EPAD = _NS * _NCH * _C
_BLK = 1024

_SC_PARAMS = pltpu.CompilerParams(use_tc_tiling_on_sc=False)


def _make_sc_apply():
    mesh = plsc.VectorSubcoreMesh(
        core_axis_name="c", subcore_axis_name="s",
        num_cores=_NC, num_subcores=_NS)
    rows_per = _NP // _NS
    n_cp = rows_per // _C

    def body(u_hbm, gidx_hbm, sidx_hbm, out_hbm, gidx_v, sidx_v,
             bbuf0, bbuf1, bbuf2, bbuf3, fbuf0, fbuf1, y_sh, *sems):
        bbufs = (bbuf0, bbuf1, bbuf2, bbuf3)
        fbufs = (fbuf0, fbuf1)
        gsem = sems[:_NBUF]
        ssem = sems[_NBUF:]
        cid = lax.axis_index("c")
        sid = lax.axis_index("s")
        pltpu.sync_copy(gidx_hbm.at[cid, sid], gidx_v)
        pltpu.sync_copy(sidx_hbm.at[sid], sidx_v)

        zero = jnp.zeros((16,), jnp.float32)

        def zrow(i, carry):
            for j in range(_DH // 16):
                fbuf0[i, pl.ds(j * 16, 16)] = zero
                fbuf1[i, pl.ds(j * 16, 16)] = zero
            return carry

        lax.fori_loop(0, _C, zrow, 0)
        base = sid * rows_per

        def zcp(i, carry):
            pltpu.sync_copy(fbuf0, y_sh.at[pl.ds(base + i * _C, _C)])
            return carry

        lax.fori_loop(0, n_cp, zcp, 0)
        plsc.subcore_barrier()

        shift16 = jnp.full((16,), 16, jnp.int32)
        mask_hi = jnp.full((16,), -65536, jnp.int32)

        for b in range(_NBUF):
            pltpu.async_copy(u_hbm.at[gidx_v.at[b]], bbufs[b], gsem[b])
        for b in range(2):
            pltpu.async_copy(
                fbufs[b], y_sh.at[sidx_v.at[0]], ssem[b], add=True)

        def grp(i, carry):
            for b in range(_NBUF):
                j = _NBUF * i + b
                jn = jnp.minimum(j + _NBUF, _NCH - 1)
                fb = fbufs[b % 2]
                pltpu.make_async_copy(
                    u_hbm.at[gidx_v.at[j]], bbufs[b], gsem[b]).wait()
                pltpu.make_async_copy(
                    fb, y_sh.at[sidx_v.at[j]], ssem[b % 2]).wait()

                def crow(r, carry2):
                    for q in range(2):
                        w = bbufs[b][r, pl.ds(16 * q, 16)]
                        fb[r, pl.ds(32 * q, 16)] = lax.bitcast_convert_type(
                            lax.shift_left(w, shift16), jnp.float32)
                        fb[r, pl.ds(32 * q + 16, 16)] = lax.bitcast_convert_type(
                            lax.bitwise_and(w, mask_hi), jnp.float32)
                    return carry2

                lax.fori_loop(0, _C, crow, 0)
                pltpu.async_copy(
                    fb, y_sh.at[sidx_v.at[j]], ssem[b % 2], add=True)
                pltpu.async_copy(u_hbm.at[gidx_v.at[jn]], bbufs[b], gsem[b])
            return carry

        lax.fori_loop(0, _NCH // _NBUF, grp, 0)
        for b in range(2):
            pltpu.make_async_copy(
                fbufs[b], y_sh.at[sidx_v.at[_NCH - 1]], ssem[b]).wait()
        for b in range(_NBUF):
            pltpu.make_async_copy(
                u_hbm.at[gidx_v.at[_NCH - 1]], bbufs[b], gsem[b]).wait()
        plsc.subcore_barrier()

        def ocp(i, carry):
            pltpu.sync_copy(y_sh.at[pl.ds(base + i * _C, _C)],
                            out_hbm.at[cid, pl.ds(base + i * _C, _C)])
            return carry

        lax.fori_loop(0, n_cp, ocp, 0)

    return pl.kernel(
        body,
        out_type=jax.ShapeDtypeStruct((_NC, _NP, _DH), jnp.float32),
        mesh=mesh,
        scratch_types=[
            pltpu.VMEM((_NCH, _C), jnp.int32),
            pltpu.VMEM((_NCH, _C), jnp.int32),
            pltpu.VMEM((_C, _DH // 2), jnp.int32),
            pltpu.VMEM((_C, _DH // 2), jnp.int32),
            pltpu.VMEM((_C, _DH // 2), jnp.int32),
            pltpu.VMEM((_C, _DH // 2), jnp.int32),
            pltpu.VMEM((_C, _DH), jnp.float32),
            pltpu.VMEM((_C, _DH), jnp.float32),
            pltpu.VMEM_SHARED((_NP, _DH), jnp.float32),
        ] + [pltpu.SemaphoreType.DMA] * (_NBUF + 2),
        compiler_params=_SC_PARAMS,
        name="sc_adj_apply",
    )


def _make_sc_deg():
    mesh = plsc.VectorSubcoreMesh(
        core_axis_name="c", subcore_axis_name="s",
        num_cores=_NC, num_subcores=_NS)
    _W = 16
    rows_per = _NP // _NS
    n_cp = rows_per // _C

    def body(idx_hbm, out_hbm, idx_v, buf, y_sh, *sems):
        cid = lax.axis_index("c")
        sid = lax.axis_index("s")
        pltpu.sync_copy(idx_hbm.at[cid, sid], idx_v)

        one = jnp.ones((16,), jnp.float32)
        zero = jnp.zeros((16,), jnp.float32)

        def orow(i, carry):
            buf[i, pl.ds(0, 16)] = one
            return carry

        lax.fori_loop(0, _C, orow, 0)
        base = sid * rows_per

        def zrow(i, carry):
            buf[_C + i, pl.ds(0, 16)] = zero
            return carry

        lax.fori_loop(0, _C, zrow, 0)

        def zcp(i, carry):
            pltpu.sync_copy(buf.at[pl.ds(_C, _C)],
                            y_sh.at[pl.ds(base + i * _C, _C)])
            return carry

        lax.fori_loop(0, n_cp, zcp, 0)
        plsc.subcore_barrier()

        ones_src = buf.at[pl.ds(0, _C)]

        def grp(i, carry):
            for b in range(_NBUF):
                j = _NBUF * i + b
                pltpu.async_copy(
                    ones_src, y_sh.at[idx_v.at[j]], sems[b], add=True)
            for b in range(_NBUF):
                j = _NBUF * i + b
                pltpu.make_async_copy(
                    ones_src, y_sh.at[idx_v.at[j]], sems[b]).wait()
            return carry

        lax.fori_loop(0, _NCH // _NBUF, grp, 0)
        plsc.subcore_barrier()

        def ocp(i, carry):
            pltpu.sync_copy(y_sh.at[pl.ds(base + i * _C, _C)],
                            out_hbm.at[cid, pl.ds(base + i * _C, _C)])
            return carry

        lax.fori_loop(0, n_cp, ocp, 0)

    return pl.kernel(
        body,
        out_type=jax.ShapeDtypeStruct((_NC, _NP, _W), jnp.float32),
        mesh=mesh,
        scratch_types=[
            pltpu.VMEM((_NCH, _C), jnp.int32),
            pltpu.VMEM((2 * _C, _W), jnp.float32),
            pltpu.VMEM_SHARED((_NP, _W), jnp.float32),
        ] + [pltpu.SemaphoreType.DMA] * _NBUF,
        compiler_params=_SC_PARAMS,
        name="sc_deg",
    )


_sc_apply = _make_sc_apply()
_sc_deg = _make_sc_deg()


def _dinv(degc):
    def body(a_ref, o_ref):
        deg = a_ref[...]
        node = (lax.broadcasted_iota(jnp.int32, (80, 128), 0) * 128
                + lax.broadcasted_iota(jnp.int32, (80, 128), 1))
        ok = (deg > 0.5) & (node < _N)
        o_ref[...] = jnp.where(ok, lax.rsqrt(jnp.maximum(deg, 1.0)), 0.0)

    return pl.pallas_call(
        body,
        out_shape=jax.ShapeDtypeStruct((80, 128), jnp.float32),
    )(degc)


def _uperm():
    col = lax.broadcasted_iota(jnp.int32, (_D, _D), 1)
    r = col % 32
    fcol = (col // 32) * 32 + jnp.where(r % 2 == 0, r // 2, 16 + r // 2)
    row = lax.broadcasted_iota(jnp.int32, (_D, _D), 0)
    return (row == fcol).astype(jnp.float32)


def _step0(xin, dinvb, W, F):
    def body(x_ref, d_ref, w_ref, u_ref, a_ref):
        xv = x_ref[...]
        uu = jnp.dot(d_ref[...] * xv, _uperm(),
                     preferred_element_type=jnp.float32).astype(jnp.bfloat16)
        u_ref[0] = uu[:, :_DH]
        u_ref[1] = uu[:, _DH:]
        a_ref[...] = jnp.dot(xv, w_ref[...], preferred_element_type=jnp.float32)

    return pl.pallas_call(
        body,
        grid=(_NP // _BLK,),
        in_specs=[
            pl.BlockSpec((_BLK, _D), lambda i: (i, 0)),
            pl.BlockSpec((_BLK, _D), lambda i: (i, 0)),
            pl.BlockSpec((_D, F), lambda i: (0, 0)),
        ],
        out_specs=[
            pl.BlockSpec((_NC, _BLK, _DH), lambda i: (0, i, 0)),
            pl.BlockSpec((_BLK, F), lambda i: (i, 0)),
        ],
        out_shape=[
            jax.ShapeDtypeStruct((_NC, _NP, _DH), jnp.bfloat16),
            jax.ShapeDtypeStruct((_NP, F), jnp.float32),
        ],
    )(xin, dinvb, W)


def _stepk(z, dinvb, txprev, W, acc, c1, c2, F):
    def body(z_ref, d_ref, p_ref, w_ref, ain_ref,
             tx_ref, u_ref, aout_ref):
        zz = jnp.concatenate([z_ref[0], z_ref[1]], axis=1)
        tx = c1 * d_ref[...] * zz + c2 * p_ref[...]
        tx_ref[...] = tx
        uu = jnp.dot(d_ref[...] * tx, _uperm(),
                     preferred_element_type=jnp.float32).astype(jnp.bfloat16)
        u_ref[0] = uu[:, :_DH]
        u_ref[1] = uu[:, _DH:]
        aout_ref[...] = ain_ref[...] + jnp.dot(
            tx, w_ref[...], preferred_element_type=jnp.float32)

    return pl.pallas_call(
        body,
        grid=(_NP // _BLK,),
        in_specs=[
            pl.BlockSpec((_NC, _BLK, _DH), lambda i: (0, i, 0)),
            pl.BlockSpec((_BLK, _D), lambda i: (i, 0)),
            pl.BlockSpec((_BLK, _D), lambda i: (i, 0)),
            pl.BlockSpec((_D, F), lambda i: (0, 0)),
            pl.BlockSpec((_BLK, F), lambda i: (i, 0)),
        ],
        out_specs=[
            pl.BlockSpec((_BLK, _D), lambda i: (i, 0)),
            pl.BlockSpec((_NC, _BLK, _DH), lambda i: (0, i, 0)),
            pl.BlockSpec((_BLK, F), lambda i: (i, 0)),
        ],
        out_shape=[
            jax.ShapeDtypeStruct((_NP, _D), jnp.float32),
            jax.ShapeDtypeStruct((_NC, _NP, _DH), jnp.bfloat16),
            jax.ShapeDtypeStruct((_NP, F), jnp.float32),
        ],
    )(z, dinvb, txprev, W, acc)


def _steplast(z, dinvb, txprev, W, b, acc, F):
    def body(z_ref, d_ref, p_ref, w_ref, b_ref, ain_ref, o_ref):
        zz = jnp.concatenate([z_ref[0], z_ref[1]], axis=1)
        tx = -2.0 * d_ref[...] * zz - p_ref[...]
        o_ref[...] = jnp.maximum(
            ain_ref[...]
            + jnp.dot(tx, w_ref[...], preferred_element_type=jnp.float32)
            + b_ref[...], 0.0)

    return pl.pallas_call(
        body,
        grid=(_NP // _BLK,),
        in_specs=[
            pl.BlockSpec((_NC, _BLK, _DH), lambda i: (0, i, 0)),
            pl.BlockSpec((_BLK, _D), lambda i: (i, 0)),
            pl.BlockSpec((_BLK, _D), lambda i: (i, 0)),
            pl.BlockSpec((_D, F), lambda i: (0, 0)),
            pl.BlockSpec((1, F), lambda i: (0, 0)),
            pl.BlockSpec((_BLK, F), lambda i: (i, 0)),
        ],
        out_specs=pl.BlockSpec((_BLK, F), lambda i: (i, 0)),
        out_shape=jax.ShapeDtypeStruct((_NP, F), jnp.float32),
    )(z, dinvb, txprev, W, b, acc)


def _pool(H, batchf):
    def body(b_ref, h_ref, s_ref, c_ref):
        i = pl.program_id(0)
        oh = (b_ref[...] == lax.broadcasted_iota(
            jnp.int32, (_BLK, _G), 1).astype(jnp.float32)).astype(jnp.float32)
        psum = lax.dot_general(oh, h_ref[...], (((0,), (0,)), ((), ())),
                               preferred_element_type=jnp.float32)
        pcnt = jnp.broadcast_to(jnp.sum(oh, axis=0)[:, None], (_G, 128))

        @pl.when(i == 0)
        def _():
            s_ref[...] = jnp.zeros_like(s_ref)
            c_ref[...] = jnp.zeros_like(c_ref)

        s_ref[...] += psum
        c_ref[...] += pcnt

    return pl.pallas_call(
        body,
        grid=(_NP // _BLK,),
        in_specs=[
            pl.BlockSpec((_BLK, 1), lambda i: (i, 0)),
            pl.BlockSpec((_BLK, 512), lambda i: (i, 0)),
        ],
        out_specs=[
            pl.BlockSpec((_G, 512), lambda i: (0, 0)),
            pl.BlockSpec((_G, 128), lambda i: (0, 0)),
        ],
        out_shape=[
            jax.ShapeDtypeStruct((_G, 512), jnp.float32),
            jax.ShapeDtypeStruct((_G, 128), jnp.float32),
        ],
    )(batchf, H)


def _head(sums, cnt, Wfc, bfc, cmf):
    def body(s_ref, c_ref, w_ref, b_ref, cm_ref, o_ref):
        counts = jnp.maximum(c_ref[...][:, 0:1], 1.0)
        pooled = s_ref[...] / counts
        logits = jnp.dot(pooled, w_ref[...],
                         preferred_element_type=jnp.float32) + b_ref[...]
        cmcol = jnp.reshape(cm_ref[...], (_NO, 1))
        P = (cmcol == lax.broadcasted_iota(
            jnp.int32, (_NO, _NB), 1).astype(jnp.float32)).astype(jnp.float32)
        seg = jnp.log(jnp.dot(jnp.exp(logits), P,
                              preferred_element_type=jnp.float32))
        norm = lax.dot_general(seg, P, (((1,), (1,)), ((), ())),
                               preferred_element_type=jnp.float32)
        o_ref[...] = logits - norm

    return pl.pallas_call(
        body,
        out_shape=jax.ShapeDtypeStruct((_G, _NO), jnp.float32),
    )(sums, cnt, Wfc, bfc, cmf)


def _as_i32(u):
    return lax.bitcast_convert_type(
        u.reshape(_NC * _NP, _DH // 2, 2), jnp.int32)


def _conv(xin, dinvb, gidx, sidx, W, b2, F):
    u0, acc = _step0(xin, dinvb, W[0], F)
    z = _sc_apply(_as_i32(u0), gidx, sidx)
    tx1, u1, acc = _stepk(z, dinvb, xin, W[1], acc, -1.0, 0.0, F)
    z = _sc_apply(_as_i32(u1), gidx, sidx)
    tx2, u2, acc = _stepk(z, dinvb, xin, W[2], acc, -2.0, -1.0, F)
    z = _sc_apply(_as_i32(u2), gidx, sidx)
    tx3, u3, acc = _stepk(z, dinvb, tx1, W[3], acc, -2.0, -1.0, F)
    z = _sc_apply(_as_i32(u3), gidx, sidx)
    return _steplast(z, dinvb, tx2, W[4], b2, acc, F)


def kernel(x, edge_index, batch, class_mask,
           W11, b11, W12, b12, W21, b21, W22, b22, Wfc, bfc):
    f32 = jnp.float32
    xp = jnp.pad(x, ((0, _NP - _N), (0, 0)))

    pad = _EPAD - _E
    sink = jnp.full((pad,), _NP - 1, jnp.int32)
    g_f = jnp.concatenate([edge_index[0], sink]).reshape(_NS, _NCH, _C)
    s_f = jnp.concatenate([edge_index[1], sink]).reshape(_NS, _NCH, _C)
    g_f2 = jnp.stack([g_f, g_f + _NP])
    s_f2 = jnp.stack([s_f, s_f + _NP])

    cnt = _sc_deg(jnp.stack([s_f, g_f]))
    d_f = _dinv(cnt[1, :, 0].reshape(80, 128))
    d_r = _dinv(cnt[0, :, 0].reshape(80, 128))
    dinvb_f = jnp.broadcast_to(d_f.reshape(_NP, 1), (_NP, _D))
    dinvb_r = jnp.broadcast_to(d_r.reshape(_NP, 1), (_NP, _D))

    x1 = _conv(xp, dinvb_f, g_f2, s_f, W11, b11.reshape(1, 64), 64)
    x2 = _conv(xp, dinvb_r, s_f2, g_f, W12, b12.reshape(1, 64), 64)
    h = jnp.concatenate([x1, x2], axis=1)
    y1 = _conv(h, dinvb_f, g_f2, s_f, W21, b21.reshape(1, 256), 256)
    y2 = _conv(h, dinvb_r, s_f2, g_f, W22, b22.reshape(1, 256), 256)
    H = jnp.concatenate([y1, y2], axis=1)

    batchf = jnp.pad(batch, (0, _NP - _N), constant_values=_G)
    batchf = batchf.astype(f32).reshape(_NP, 1)
    sums, cnt2 = _pool(H, batchf)
    return _head(sums, cnt2, Wfc, bfc.reshape(1, _NO),
                 class_mask.astype(f32).reshape(1, _NO))


# speedup vs baseline: 3.4486x; 1.0382x over previous
import jax
import jax.numpy as jnp
from jax import lax
from jax.experimental import pallas as pl
from jax.experimental.pallas import tpu as pltpu
from jax.experimental.pallas import tpu_sc as plsc

_N = 10000
_E = 320000
_D = 128
_G = 64
_NB = 16
_NO = 128
_NP = 10240
_NC = 2
_NS = 16
_C = 128
_NCH = 160
_NBUF = 4
_DH = 64
_EPAD = _NS * _NCH * _C
_BLK = 1024

_SC_PARAMS = pltpu.CompilerParams(use_tc_tiling_on_sc=False)


def _make_sc_apply():
    mesh = plsc.VectorSubcoreMesh(
        core_axis_name="c", subcore_axis_name="s",
        num_cores=_NC, num_subcores=_NS)
    rows_per = _NP // _NS
    n_cp = rows_per // _C

    def body(u_hbm, gidx_hbm, sidx_hbm, out_hbm, gidx_v, sidx_v,
             bbuf0, bbuf1, bbuf2, bbuf3, fbuf0, fbuf1, y_sh, *sems):
        bbufs = (bbuf0, bbuf1, bbuf2, bbuf3)
        fbufs = (fbuf0, fbuf1)
        gsem = sems[:_NBUF]
        ssem = sems[_NBUF:]
        cid = lax.axis_index("c")
        sid = lax.axis_index("s")
        pltpu.sync_copy(gidx_hbm.at[cid, sid], gidx_v)
        pltpu.sync_copy(sidx_hbm.at[sid], sidx_v)

        zero = jnp.zeros((16,), jnp.float32)

        def zrow(i, carry):
            for j in range(_DH // 16):
                fbuf0[i, pl.ds(j * 16, 16)] = zero
                fbuf1[i, pl.ds(j * 16, 16)] = zero
            return carry

        lax.fori_loop(0, _C, zrow, 0)
        base = sid * rows_per

        def zcp(i, carry):
            pltpu.sync_copy(fbuf0, y_sh.at[pl.ds(base + i * _C, _C)])
            return carry

        lax.fori_loop(0, n_cp, zcp, 0)
        plsc.subcore_barrier()

        shift16 = jnp.full((16,), 16, jnp.int32)

        for b in range(_NBUF):
            pltpu.async_copy(u_hbm.at[gidx_v.at[b]], bbufs[b], gsem[b])
        for b in range(2):
            pltpu.async_copy(
                fbufs[b], y_sh.at[sidx_v.at[0]], ssem[b], add=True)

        def grp(i, carry):
            for b in range(_NBUF):
                j = _NBUF * i + b
                jn = jnp.minimum(j + _NBUF, _NCH - 1)
                fb = fbufs[b % 2]
                pltpu.make_async_copy(
                    u_hbm.at[gidx_v.at[j]], bbufs[b], gsem[b]).wait()
                pltpu.make_async_copy(
                    fb, y_sh.at[sidx_v.at[j]], ssem[b % 2]).wait()

                def crow(r, carry2):
                    for q in range(2):
                        w = bbufs[b][r, pl.ds(16 * q, 16)]
                        fb[r, pl.ds(32 * q, 16)] = lax.bitcast_convert_type(
                            lax.shift_left(w, shift16), jnp.float32)
                        fb[r, pl.ds(32 * q + 16, 16)] = lax.bitcast_convert_type(
                            w, jnp.float32)
                    return carry2

                lax.fori_loop(0, _C, crow, 0, unroll=4)
                pltpu.async_copy(
                    fb, y_sh.at[sidx_v.at[j]], ssem[b % 2], add=True)
                pltpu.async_copy(u_hbm.at[gidx_v.at[jn]], bbufs[b], gsem[b])
            return carry

        lax.fori_loop(0, _NCH // _NBUF, grp, 0)
        for b in range(2):
            pltpu.make_async_copy(
                fbufs[b], y_sh.at[sidx_v.at[_NCH - 1]], ssem[b]).wait()
        for b in range(_NBUF):
            pltpu.make_async_copy(
                u_hbm.at[gidx_v.at[_NCH - 1]], bbufs[b], gsem[b]).wait()
        plsc.subcore_barrier()

        def ocp(i, carry):
            pltpu.sync_copy(y_sh.at[pl.ds(base + i * _C, _C)],
                            out_hbm.at[cid, pl.ds(base + i * _C, _C)])
            return carry

        lax.fori_loop(0, n_cp, ocp, 0)

    return pl.kernel(
        body,
        out_type=jax.ShapeDtypeStruct((_NC, _NP, _DH), jnp.float32),
        mesh=mesh,
        scratch_types=[
            pltpu.VMEM((_NCH, _C), jnp.int32),
            pltpu.VMEM((_NCH, _C), jnp.int32),
            pltpu.VMEM((_C, _DH // 2), jnp.int32),
            pltpu.VMEM((_C, _DH // 2), jnp.int32),
            pltpu.VMEM((_C, _DH // 2), jnp.int32),
            pltpu.VMEM((_C, _DH // 2), jnp.int32),
            pltpu.VMEM((_C, _DH), jnp.float32),
            pltpu.VMEM((_C, _DH), jnp.float32),
            pltpu.VMEM_SHARED((_NP, _DH), jnp.float32),
        ] + [pltpu.SemaphoreType.DMA] * (_NBUF + 2),
        compiler_params=_SC_PARAMS,
        name="sc_adj_apply",
    )


def _make_sc_deg():
    mesh = plsc.VectorSubcoreMesh(
        core_axis_name="c", subcore_axis_name="s",
        num_cores=_NC, num_subcores=_NS)
    _W = 16
    rows_per = _NP // _NS
    n_cp = rows_per // _C

    def body(idx_hbm, out_hbm, idx_v, buf, y_sh, *sems):
        cid = lax.axis_index("c")
        sid = lax.axis_index("s")
        pltpu.sync_copy(idx_hbm.at[cid, sid], idx_v)

        one = jnp.ones((16,), jnp.float32)
        zero = jnp.zeros((16,), jnp.float32)

        def orow(i, carry):
            buf[i, pl.ds(0, 16)] = one
            return carry

        lax.fori_loop(0, _C, orow, 0)
        base = sid * rows_per

        def zrow(i, carry):
            buf[_C + i, pl.ds(0, 16)] = zero
            return carry

        lax.fori_loop(0, _C, zrow, 0)

        def zcp(i, carry):
            pltpu.sync_copy(buf.at[pl.ds(_C, _C)],
                            y_sh.at[pl.ds(base + i * _C, _C)])
            return carry

        lax.fori_loop(0, n_cp, zcp, 0)
        plsc.subcore_barrier()

        ones_src = buf.at[pl.ds(0, _C)]

        def grp(i, carry):
            for b in range(_NBUF):
                j = _NBUF * i + b
                pltpu.async_copy(
                    ones_src, y_sh.at[idx_v.at[j]], sems[b], add=True)
            for b in range(_NBUF):
                j = _NBUF * i + b
                pltpu.make_async_copy(
                    ones_src, y_sh.at[idx_v.at[j]], sems[b]).wait()
            return carry

        lax.fori_loop(0, _NCH // _NBUF, grp, 0)
        plsc.subcore_barrier()

        def ocp(i, carry):
            pltpu.sync_copy(y_sh.at[pl.ds(base + i * _C, _C)],
                            out_hbm.at[cid, pl.ds(base + i * _C, _C)])
            return carry

        lax.fori_loop(0, n_cp, ocp, 0)

    return pl.kernel(
        body,
        out_type=jax.ShapeDtypeStruct((_NC, _NP, _W), jnp.float32),
        mesh=mesh,
        scratch_types=[
            pltpu.VMEM((_NCH, _C), jnp.int32),
            pltpu.VMEM((2 * _C, _W), jnp.float32),
            pltpu.VMEM_SHARED((_NP, _W), jnp.float32),
        ] + [pltpu.SemaphoreType.DMA] * _NBUF,
        compiler_params=_SC_PARAMS,
        name="sc_deg",
    )


_sc_apply = _make_sc_apply()
_sc_deg = _make_sc_deg()


def _dinv(degc):
    def body(a_ref, o_ref):
        deg = a_ref[...]
        node = (lax.broadcasted_iota(jnp.int32, (80, 128), 0) * 128
                + lax.broadcasted_iota(jnp.int32, (80, 128), 1))
        ok = (deg > 0.5) & (node < _N)
        o_ref[...] = jnp.where(ok, lax.rsqrt(jnp.maximum(deg, 1.0)), 0.0)

    return pl.pallas_call(
        body,
        out_shape=jax.ShapeDtypeStruct((80, 128), jnp.float32),
    )(degc)


def _uperm():
    col = lax.broadcasted_iota(jnp.int32, (_D, _D), 1)
    r = col % 32
    fcol = (col // 32) * 32 + jnp.where(r % 2 == 0, r // 2, 16 + r // 2)
    row = lax.broadcasted_iota(jnp.int32, (_D, _D), 0)
    return (row == fcol).astype(jnp.float32)


def _step0(xin, dinvb, W, F):
    def body(x_ref, d_ref, w_ref, u_ref, a_ref):
        xv = x_ref[...]
        uu = jnp.dot(d_ref[...] * xv, _uperm(),
                     preferred_element_type=jnp.float32).astype(jnp.bfloat16)
        u_ref[0] = uu[:, :_DH]
        u_ref[1] = uu[:, _DH:]
        a_ref[...] = jnp.dot(xv, w_ref[...], preferred_element_type=jnp.float32)

    return pl.pallas_call(
        body,
        grid=(_NP // _BLK,),
        in_specs=[
            pl.BlockSpec((_BLK, _D), lambda i: (i, 0)),
            pl.BlockSpec((_BLK, _D), lambda i: (i, 0)),
            pl.BlockSpec((_D, F), lambda i: (0, 0)),
        ],
        out_specs=[
            pl.BlockSpec((_NC, _BLK, _DH), lambda i: (0, i, 0)),
            pl.BlockSpec((_BLK, F), lambda i: (i, 0)),
        ],
        out_shape=[
            jax.ShapeDtypeStruct((_NC, _NP, _DH), jnp.bfloat16),
            jax.ShapeDtypeStruct((_NP, F), jnp.float32),
        ],
    )(xin, dinvb, W)


def _stepk(z, dinvb, txprev, W, acc, c1, c2, F):
    def body(z_ref, d_ref, p_ref, w_ref, ain_ref,
             tx_ref, u_ref, aout_ref):
        zz = jnp.concatenate([z_ref[0], z_ref[1]], axis=1)
        tx = c1 * d_ref[...] * zz + c2 * p_ref[...]
        tx_ref[...] = tx
        uu = jnp.dot(d_ref[...] * tx, _uperm(),
                     preferred_element_type=jnp.float32).astype(jnp.bfloat16)
        u_ref[0] = uu[:, :_DH]
        u_ref[1] = uu[:, _DH:]
        aout_ref[...] = ain_ref[...] + jnp.dot(
            tx, w_ref[...], preferred_element_type=jnp.float32)

    return pl.pallas_call(
        body,
        grid=(_NP // _BLK,),
        in_specs=[
            pl.BlockSpec((_NC, _BLK, _DH), lambda i: (0, i, 0)),
            pl.BlockSpec((_BLK, _D), lambda i: (i, 0)),
            pl.BlockSpec((_BLK, _D), lambda i: (i, 0)),
            pl.BlockSpec((_D, F), lambda i: (0, 0)),
            pl.BlockSpec((_BLK, F), lambda i: (i, 0)),
        ],
        out_specs=[
            pl.BlockSpec((_BLK, _D), lambda i: (i, 0)),
            pl.BlockSpec((_NC, _BLK, _DH), lambda i: (0, i, 0)),
            pl.BlockSpec((_BLK, F), lambda i: (i, 0)),
        ],
        out_shape=[
            jax.ShapeDtypeStruct((_NP, _D), jnp.float32),
            jax.ShapeDtypeStruct((_NC, _NP, _DH), jnp.bfloat16),
            jax.ShapeDtypeStruct((_NP, F), jnp.float32),
        ],
    )(z, dinvb, txprev, W, acc)


def _steplast(z, dinvb, txprev, W, b, acc, F):
    def body(z_ref, d_ref, p_ref, w_ref, b_ref, ain_ref, o_ref):
        zz = jnp.concatenate([z_ref[0], z_ref[1]], axis=1)
        tx = -2.0 * d_ref[...] * zz - p_ref[...]
        o_ref[...] = jnp.maximum(
            ain_ref[...]
            + jnp.dot(tx, w_ref[...], preferred_element_type=jnp.float32)
            + b_ref[...], 0.0)

    return pl.pallas_call(
        body,
        grid=(_NP // _BLK,),
        in_specs=[
            pl.BlockSpec((_NC, _BLK, _DH), lambda i: (0, i, 0)),
            pl.BlockSpec((_BLK, _D), lambda i: (i, 0)),
            pl.BlockSpec((_BLK, _D), lambda i: (i, 0)),
            pl.BlockSpec((_D, F), lambda i: (0, 0)),
            pl.BlockSpec((1, F), lambda i: (0, 0)),
            pl.BlockSpec((_BLK, F), lambda i: (i, 0)),
        ],
        out_specs=pl.BlockSpec((_BLK, F), lambda i: (i, 0)),
        out_shape=jax.ShapeDtypeStruct((_NP, F), jnp.float32),
    )(z, dinvb, txprev, W, b, acc)


def _pool(H, batchf):
    def body(b_ref, h_ref, s_ref, c_ref):
        i = pl.program_id(0)
        oh = (b_ref[...] == lax.broadcasted_iota(
            jnp.int32, (_BLK, _G), 1).astype(jnp.float32)).astype(jnp.float32)
        psum = lax.dot_general(oh, h_ref[...], (((0,), (0,)), ((), ())),
                               preferred_element_type=jnp.float32)
        pcnt = jnp.broadcast_to(jnp.sum(oh, axis=0)[:, None], (_G, 128))

        @pl.when(i == 0)
        def _():
            s_ref[...] = jnp.zeros_like(s_ref)
            c_ref[...] = jnp.zeros_like(c_ref)

        s_ref[...] += psum
        c_ref[...] += pcnt

    return pl.pallas_call(
        body,
        grid=(_NP // _BLK,),
        in_specs=[
            pl.BlockSpec((_BLK, 1), lambda i: (i, 0)),
            pl.BlockSpec((_BLK, 512), lambda i: (i, 0)),
        ],
        out_specs=[
            pl.BlockSpec((_G, 512), lambda i: (0, 0)),
            pl.BlockSpec((_G, 128), lambda i: (0, 0)),
        ],
        out_shape=[
            jax.ShapeDtypeStruct((_G, 512), jnp.float32),
            jax.ShapeDtypeStruct((_G, 128), jnp.float32),
        ],
    )(batchf, H)


def _head(sums, cnt, Wfc, bfc, cmf):
    def body(s_ref, c_ref, w_ref, b_ref, cm_ref, o_ref):
        counts = jnp.maximum(c_ref[...][:, 0:1], 1.0)
        pooled = s_ref[...] / counts
        logits = jnp.dot(pooled, w_ref[...],
                         preferred_element_type=jnp.float32) + b_ref[...]
        cmcol = jnp.reshape(cm_ref[...], (_NO, 1))
        P = (cmcol == lax.broadcasted_iota(
            jnp.int32, (_NO, _NB), 1).astype(jnp.float32)).astype(jnp.float32)
        seg = jnp.log(jnp.dot(jnp.exp(logits), P,
                              preferred_element_type=jnp.float32))
        norm = lax.dot_general(seg, P, (((1,), (1,)), ((), ())),
                               preferred_element_type=jnp.float32)
        o_ref[...] = logits - norm

    return pl.pallas_call(
        body,
        out_shape=jax.ShapeDtypeStruct((_G, _NO), jnp.float32),
    )(sums, cnt, Wfc, bfc, cmf)


def _as_i32(u):
    return lax.bitcast_convert_type(
        u.reshape(_NC * _NP, _DH // 2, 2), jnp.int32)


def _conv(xin, dinvb, gidx, sidx, W, b2, F):
    u0, acc = _step0(xin, dinvb, W[0], F)
    z = _sc_apply(_as_i32(u0), gidx, sidx)
    tx1, u1, acc = _stepk(z, dinvb, xin, W[1], acc, -1.0, 0.0, F)
    z = _sc_apply(_as_i32(u1), gidx, sidx)
    tx2, u2, acc = _stepk(z, dinvb, xin, W[2], acc, -2.0, -1.0, F)
    z = _sc_apply(_as_i32(u2), gidx, sidx)
    tx3, u3, acc = _stepk(z, dinvb, tx1, W[3], acc, -2.0, -1.0, F)
    z = _sc_apply(_as_i32(u3), gidx, sidx)
    return _steplast(z, dinvb, tx2, W[4], b2, acc, F)


def kernel(x, edge_index, batch, class_mask,
           W11, b11, W12, b12, W21, b21, W22, b22, Wfc, bfc):
    f32 = jnp.float32
    xp = jnp.pad(x, ((0, _NP - _N), (0, 0)))

    pad = _EPAD - _E
    sink = jnp.full((pad,), _NP - 1, jnp.int32)
    g_f = jnp.concatenate([edge_index[0], sink]).reshape(_NS, _NCH, _C)
    s_f = jnp.concatenate([edge_index[1], sink]).reshape(_NS, _NCH, _C)
    g_f2 = jnp.stack([g_f, g_f + _NP])
    s_f2 = jnp.stack([s_f, s_f + _NP])

    cnt = _sc_deg(jnp.stack([s_f, g_f]))
    d_f = _dinv(cnt[1, :, 0].reshape(80, 128))
    d_r = _dinv(cnt[0, :, 0].reshape(80, 128))
    dinvb_f = jnp.broadcast_to(d_f.reshape(_NP, 1), (_NP, _D))
    dinvb_r = jnp.broadcast_to(d_r.reshape(_NP, 1), (_NP, _D))

    x1 = _conv(xp, dinvb_f, g_f2, s_f, W11, b11.reshape(1, 64), 64)
    x2 = _conv(xp, dinvb_r, s_f2, g_f, W12, b12.reshape(1, 64), 64)
    h = jnp.concatenate([x1, x2], axis=1)
    y1 = _conv(h, dinvb_f, g_f2, s_f, W21, b21.reshape(1, 256), 256)
    y2 = _conv(h, dinvb_r, s_f2, g_f, W22, b22.reshape(1, 256), 256)
    H = jnp.concatenate([y1, y2], axis=1)

    batchf = jnp.pad(batch, (0, _NP - _N), constant_values=_G)
    batchf = batchf.astype(f32).reshape(_NP, 1)
    sums, cnt2 = _pool(H, batchf)
    return _head(sums, cnt2, Wfc, bfc.reshape(1, _NO),
                 class_mask.astype(f32).reshape(1, _NO))
